# baseline (speedup 1.0000x reference)
import functools

import jax
import jax.numpy as jnp
from jax import lax
from jax.experimental import pallas as pl
from jax.experimental.pallas import tpu as pltpu
from jax.experimental.pallas import tpu_sc as plsc

D = 64
NC = 2
NS = 16
NW = NC * NS
NBUF = 4
PAN = 128
L = 16


@functools.lru_cache(maxsize=None)
def _build(B, H, V):
    rpw = B // NW
    VMAIN = (V // PAN) * PAN
    npan = VMAIN // PAN
    neach = npan // NW
    assert rpw % NBUF == 0 and neach % 2 == 0 and V - VMAIN == PAN // 2
    mesh = plsc.VectorSubcoreMesh(core_axis_name="c", subcore_axis_name="s")

    @functools.partial(
        pl.kernel,
        out_type=jax.ShapeDtypeStruct((V, 2 * D), jnp.float32),
        mesh=mesh,
        scratch_types=[
            pltpu.VMEM((2, D, PAN), jnp.float32),
            pltpu.VMEM((2, PAN, 2 * D), jnp.float32),
            pltpu.VMEM((PAN // 4, 2 * D), jnp.float32),
            pltpu.SemaphoreType.DMA,
            pltpu.SemaphoreType.DMA,
            pltpu.SemaphoreType.DMA,
            pltpu.SemaphoreType.DMA,
        ],
        compiler_params=pltpu.CompilerParams(needs_layout_passes=False),
    )
    def xpose(tT_hbm, tail_hbm, out_hbm, in_v, out_v, tail_v, r0, r1, w0, w1):
        wid = lax.axis_index("s") * NC + lax.axis_index("c")
        rsem = [r0, r1]
        wsem = [w0, w1]

        def pan_col(i):
            return (wid + i * NW) * PAN

        def rd(i, slot):
            pltpu.async_copy(tT_hbm.at[:, pl.ds(pan_col(i), PAN)],
                             in_v.at[slot], rsem[slot])

        def rd_wait(i, slot):
            pltpu.make_async_copy(tT_hbm.at[:, pl.ds(pan_col(i), PAN)],
                                  in_v.at[slot], rsem[slot]).wait()

        def wr(i, slot):
            pltpu.async_copy(out_v.at[slot],
                             out_hbm.at[pl.ds(pan_col(i), PAN), :],
                             wsem[slot])

        def wr_wait(i, slot):
            pltpu.make_async_copy(out_v.at[slot],
                                  out_hbm.at[pl.ds(pan_col(i), PAN), :],
                                  wsem[slot]).wait()

        iotas = [lax.iota(jnp.int32, L) + j * L for j in range(D // L)]

        def transpose(slot):
            def row(r, carry):
                rs = jnp.full((L,), r, jnp.int32)
                for k in range(D // L):
                    vals = plsc.load_gather(in_v.at[slot], [iotas[k], rs])
                    out_v[slot, r, pl.ds(k * L, L)] = vals
                return carry

            lax.fori_loop(0, PAN, row, 0)

        @pl.when(wid == 0)
        def _():
            pltpu.sync_copy(tail_hbm, tail_v)
            pltpu.sync_copy(tail_v, out_hbm.at[pl.ds(VMAIN, PAN // 4), :])

        rd(0, 0)

        def step(t, carry):
            i0 = 2 * t
            rd_wait(i0, 0)

            @pl.when(i0 >= 1)
            def _():
                wr_wait(i0 - 1, 1)

            rd(i0 + 1, 1)
            transpose(0)
            wr(i0, 0)

            i1 = i0 + 1
            rd_wait(i1, 1)
            wr_wait(i0, 0)

            @pl.when(i1 + 1 < neach)
            def _():
                rd(i1 + 1, 0)

            transpose(1)
            wr(i1, 1)
            return carry

        lax.fori_loop(0, neach // 2, step, 0)
        wr_wait(neach - 1, 1)

        @pl.when(wid + neach * NW < npan)
        def _():
            rd(neach, 0)
            rd_wait(neach, 0)
            transpose(0)
            wr(neach, 0)
            wr_wait(neach, 0)

    @functools.partial(
        pl.kernel,
        out_type=jax.ShapeDtypeStruct((B, H, 2 * D), jnp.float32),
        mesh=mesh,
        scratch_types=[
            pltpu.VMEM((B // NW, H), jnp.int32),
            pltpu.VMEM((NBUF, H, D), jnp.float32),
            pltpu.SemaphoreType.DMA,
            pltpu.SemaphoreType.DMA,
            pltpu.SemaphoreType.DMA,
            pltpu.SemaphoreType.DMA,
        ],
        compiler_params=pltpu.CompilerParams(use_tc_tiling_on_sc=False),
    )
    def emb(x_hbm, table_hbm, out_hbm, idx_v, rows_v, s0, s1, s2, s3):
        wid = lax.axis_index("s") * NC + lax.axis_index("c")
        base = wid * rpw
        sems = [s0, s1, s2, s3]
        pltpu.sync_copy(x_hbm.at[pl.ds(base, rpw), :], idx_v)

        def fire(r, slot):
            pltpu.async_copy(table_hbm.at[idx_v.at[r]], rows_v.at[slot],
                             sems[slot])

        def drain_write(r, slot):
            pltpu.make_async_copy(table_hbm.at[idx_v.at[r]],
                                  rows_v.at[slot], sems[slot]).wait()
            pltpu.sync_copy(rows_v.at[slot],
                            out_hbm.at[base + r, :, pl.ds(0, D)])

        for b in range(NBUF):
            fire(b, b)

        def step(t, carry):
            r0 = t * NBUF
            for b in range(NBUF):
                drain_write(r0 + b, b)

                @pl.when(r0 + b + NBUF < rpw)
                def _():
                    fire(r0 + b + NBUF, b)
            return carry

        lax.fori_loop(0, rpw // NBUF, step, 0)

    def kern(x, table):
        tT = table.T
        tail2 = table[VMAIN:].reshape(PAN // 4, 2 * D)
        stage = xpose(tT, tail2)
        tbl2 = stage.reshape(2 * V, D)
        xi = x.astype(jnp.int32)
        x2 = jnp.where(xi < VMAIN, xi * 2, xi + VMAIN)
        out = emb(x2, tbl2)
        return out[:, :, :D]

    return kern


def kernel(x, table):
    b, h = x.shape
    return _build(b, h, table.shape[0])(x, table)


# speedup vs baseline: 1.5523x; 1.5523x over previous
import functools

import jax
import jax.numpy as jnp
from jax import lax
from jax.experimental import pallas as pl
from jax.experimental.pallas import tpu as pltpu
from jax.experimental.pallas import tpu_sc as plsc

D = 64
NC = 2
NS = 16
NW = NC * NS
NBUF = 4
PAN = 128
L = 16


@functools.lru_cache(maxsize=None)
def _build(B, H, V):
    rpw = B // NW
    VMAIN = (V // PAN) * PAN
    npan = VMAIN // PAN
    neach = npan // NW
    assert rpw % NBUF == 0 and neach % 2 == 0 and V - VMAIN == PAN // 2
    mesh = plsc.VectorSubcoreMesh(core_axis_name="c", subcore_axis_name="s")

    @functools.partial(
        pl.kernel,
        out_type=jax.ShapeDtypeStruct((V, 2 * D), jnp.float32),
        mesh=mesh,
        scratch_types=[
            pltpu.VMEM((2, D, PAN), jnp.float32),
            pltpu.VMEM((2, PAN, 2 * D), jnp.float32),
            pltpu.VMEM((PAN // 4, 2 * D), jnp.float32),
            pltpu.SemaphoreType.DMA,
            pltpu.SemaphoreType.DMA,
            pltpu.SemaphoreType.DMA,
            pltpu.SemaphoreType.DMA,
        ],
        compiler_params=pltpu.CompilerParams(needs_layout_passes=False),
    )
    def xpose(tT_hbm, tail_hbm, out_hbm, in_v, out_v, tail_v, r0, r1, w0, w1):
        wid = lax.axis_index("s") * NC + lax.axis_index("c")
        rsem = [r0, r1]
        wsem = [w0, w1]

        def pan_col(i):
            return (wid + i * NW) * PAN

        def rd(i, slot):
            pltpu.async_copy(tT_hbm.at[:, pl.ds(pan_col(i), PAN)],
                             in_v.at[slot], rsem[slot])

        def rd_wait(i, slot):
            pltpu.make_async_copy(tT_hbm.at[:, pl.ds(pan_col(i), PAN)],
                                  in_v.at[slot], rsem[slot]).wait()

        def wr(i, slot):
            pltpu.async_copy(out_v.at[slot],
                             out_hbm.at[pl.ds(pan_col(i), PAN), :],
                             wsem[slot])

        def wr_wait(i, slot):
            pltpu.make_async_copy(out_v.at[slot],
                                  out_hbm.at[pl.ds(pan_col(i), PAN), :],
                                  wsem[slot]).wait()

        iotas = [lax.iota(jnp.int32, L) + j * L for j in range(D // L)]

        def transpose(slot):
            @plsc.parallel_loop(0, PAN, unroll=8)
            def row(r):
                rs = jnp.full((L,), r, jnp.int32)
                for k in range(D // L):
                    vals = plsc.load_gather(in_v.at[slot], [iotas[k], rs])
                    out_v[slot, r, pl.ds(k * L, L)] = vals

        @pl.when(wid == 0)
        def _():
            pltpu.sync_copy(tail_hbm, tail_v)
            pltpu.sync_copy(tail_v, out_hbm.at[pl.ds(VMAIN, PAN // 4), :])

        rd(0, 0)

        def step(t, carry):
            i0 = 2 * t
            rd_wait(i0, 0)

            @pl.when(i0 >= 1)
            def _():
                wr_wait(i0 - 1, 1)

            rd(i0 + 1, 1)
            transpose(0)
            wr(i0, 0)

            i1 = i0 + 1
            rd_wait(i1, 1)
            wr_wait(i0, 0)

            @pl.when(i1 + 1 < neach)
            def _():
                rd(i1 + 1, 0)

            transpose(1)
            wr(i1, 1)
            return carry

        lax.fori_loop(0, neach // 2, step, 0)
        wr_wait(neach - 1, 1)

        @pl.when(wid + neach * NW < npan)
        def _():
            rd(neach, 0)
            rd_wait(neach, 0)
            transpose(0)
            wr(neach, 0)
            wr_wait(neach, 0)

    @functools.partial(
        pl.kernel,
        out_type=jax.ShapeDtypeStruct((B, H, 2 * D), jnp.float32),
        mesh=mesh,
        scratch_types=[
            pltpu.VMEM((B // NW, H), jnp.int32),
            pltpu.VMEM((NBUF, H, D), jnp.float32),
            pltpu.SemaphoreType.DMA,
            pltpu.SemaphoreType.DMA,
            pltpu.SemaphoreType.DMA,
            pltpu.SemaphoreType.DMA,
        ],
        compiler_params=pltpu.CompilerParams(use_tc_tiling_on_sc=False),
    )
    def emb(x_hbm, table_hbm, out_hbm, idx_v, rows_v, s0, s1, s2, s3):
        wid = lax.axis_index("s") * NC + lax.axis_index("c")
        base = wid * rpw
        sems = [s0, s1, s2, s3]
        pltpu.sync_copy(x_hbm.at[pl.ds(base, rpw), :], idx_v)

        def fire(r, slot):
            pltpu.async_copy(table_hbm.at[idx_v.at[r]], rows_v.at[slot],
                             sems[slot])

        def drain_write(r, slot):
            pltpu.make_async_copy(table_hbm.at[idx_v.at[r]],
                                  rows_v.at[slot], sems[slot]).wait()
            pltpu.sync_copy(rows_v.at[slot],
                            out_hbm.at[base + r, :, pl.ds(0, D)])

        for b in range(NBUF):
            fire(b, b)

        def step(t, carry):
            r0 = t * NBUF
            for b in range(NBUF):
                drain_write(r0 + b, b)

                @pl.when(r0 + b + NBUF < rpw)
                def _():
                    fire(r0 + b + NBUF, b)
            return carry

        lax.fori_loop(0, rpw // NBUF, step, 0)

    def kern(x, table):
        tT = table.T
        tail2 = table[VMAIN:].reshape(PAN // 4, 2 * D)
        stage = xpose(tT, tail2)
        tbl2 = stage.reshape(2 * V, D)
        xi = x.astype(jnp.int32)
        x2 = jnp.where(xi < VMAIN, xi * 2, xi + VMAIN)
        out = emb(x2, tbl2)
        return out[:, :, :D]

    return kern


def kernel(x, table):
    b, h = x.shape
    return _build(b, h, table.shape[0])(x, table)


# speedup vs baseline: 2.1311x; 1.3729x over previous
import functools

import jax
import jax.numpy as jnp
from jax import lax
from jax.experimental import pallas as pl
from jax.experimental.pallas import tpu as pltpu
from jax.experimental.pallas import tpu_sc as plsc

D = 64
NC = 2
NS = 16
NW = NC * NS
NBUF = 4


@functools.lru_cache(maxsize=None)
def _build(B, H):
    rpw = B // NW
    assert rpw % NBUF == 0
    mesh = plsc.VectorSubcoreMesh(core_axis_name="c", subcore_axis_name="s")

    @functools.partial(
        pl.kernel,
        out_type=jax.ShapeDtypeStruct((B, H, 2 * D), jnp.float32),
        mesh=mesh,
        scratch_types=[
            pltpu.VMEM((rpw, H), jnp.int32),
            pltpu.VMEM((NBUF, H, D), jnp.float32),
            pltpu.SemaphoreType.DMA,
            pltpu.SemaphoreType.DMA,
            pltpu.SemaphoreType.DMA,
            pltpu.SemaphoreType.DMA,
        ],
        compiler_params=pltpu.CompilerParams(use_tc_tiling_on_sc=False),
    )
    def emb(x_hbm, table_hbm, out_hbm, idx_v, rows_v, s0, s1, s2, s3):
        wid = lax.axis_index("s") * NC + lax.axis_index("c")
        base = wid * rpw
        sems = [s0, s1, s2, s3]

        pltpu.sync_copy(x_hbm.at[pl.ds(base, rpw), :], idx_v)

        def fire(r, slot):
            pltpu.async_copy(table_hbm.at[idx_v.at[r]], rows_v.at[slot],
                             sems[slot])

        def drain_write(r, slot):
            pltpu.make_async_copy(table_hbm.at[idx_v.at[r]],
                                  rows_v.at[slot], sems[slot]).wait()
            pltpu.sync_copy(rows_v.at[slot],
                            out_hbm.at[base + r, :, pl.ds(0, D)])

        for b in range(NBUF):
            fire(b, b)

        def step(t, carry):
            r0 = t * NBUF
            for b in range(NBUF):
                drain_write(r0 + b, b)

                @pl.when(r0 + b + NBUF < rpw)
                def _():
                    fire(r0 + b + NBUF, b)

            return carry

        lax.fori_loop(0, rpw // NBUF, step, 0)

    def kern(x, table):
        out = emb(x.astype(jnp.int32), table)
        return out[:, :, :D]

    return kern


def kernel(x, table):
    b, h = x.shape
    return _build(b, h)(x, table)
